# Initial kernel scaffold; baseline (speedup 1.0000x reference)
import jax
import jax.numpy as jnp
from jax.experimental import pallas as pl


def kernel(x, gate_w):
    raise NotImplementedError("write your pallas kernel here")


# speedup vs baseline: 1.1997x; 1.1997x over previous
import functools

import jax
import jax.numpy as jnp
from jax.experimental import pallas as pl
from jax.experimental.pallas import tpu as pltpu

D_MODEL = 4096
N_EXPERTS = 64
TOP_K = 8
AUX_W = 0.01


def _router_body(x_ref, w_ref, gates_ref, idx_ref, aux_ref, counts_ref,
                 *, blk_t, n_blk, n_tokens):
    i = pl.program_id(0)
    logits = jax.lax.dot_general(
        x_ref[...], w_ref[...],
        dimension_numbers=(((1,), (1,)), ((), ())),
        preferred_element_type=jnp.float32,
    )
    m = jnp.max(logits, axis=1, keepdims=True)
    el = jnp.exp(logits - m)

    lane = jax.lax.broadcasted_iota(jnp.int32, (blk_t, N_EXPERTS), 1)
    work = el
    vals = []
    idxs = []
    for _ in range(TOP_K):
        mj = jnp.max(work, axis=1, keepdims=True)
        ij = jnp.min(jnp.where(work == mj, lane, N_EXPERTS),
                     axis=1, keepdims=True)
        vals.append(mj)
        idxs.append(ij)
        work = jnp.where(lane == ij, -1.0, work)

    v = jnp.concatenate(vals, axis=1)
    gates_ref[...] = v / jnp.sum(v, axis=1, keepdims=True)
    idx_ref[...] = jnp.concatenate(idxs, axis=1)

    onehot = jnp.where(lane == idxs[0], 1.0, 0.0)
    partial = jnp.sum(onehot, axis=0, keepdims=True)

    @pl.when(i == 0)
    def _init():
        counts_ref[...] = partial

    @pl.when(i > 0)
    def _acc():
        counts_ref[...] += partial

    @pl.when(i == n_blk - 1)
    def _fin():
        freq = counts_ref[...] * (1.0 / n_tokens)
        diff = freq - (1.0 / N_EXPERTS)
        aux_ref[...] = AUX_W * N_EXPERTS * jnp.sum(diff * diff,
                                                   axis=(0, 1), keepdims=True)


def kernel(x, gate_w):
    b, s, d = x.shape
    n_tokens = b * s
    blk_t = 512
    n_blk = n_tokens // blk_t
    xf = x.reshape(n_tokens, d)

    gates, idx, aux = pl.pallas_call(
        functools.partial(_router_body, blk_t=blk_t, n_blk=n_blk,
                          n_tokens=n_tokens),
        grid=(n_blk,),
        in_specs=[
            pl.BlockSpec((blk_t, d), lambda i: (i, 0)),
            pl.BlockSpec((N_EXPERTS, d), lambda i: (0, 0)),
        ],
        out_specs=[
            pl.BlockSpec((blk_t, TOP_K), lambda i: (i, 0)),
            pl.BlockSpec((blk_t, TOP_K), lambda i: (i, 0)),
            pl.BlockSpec((1, 1), lambda i: (0, 0)),
        ],
        out_shape=[
            jax.ShapeDtypeStruct((n_tokens, TOP_K), jnp.float32),
            jax.ShapeDtypeStruct((n_tokens, TOP_K), jnp.int32),
            jax.ShapeDtypeStruct((1, 1), jnp.float32),
        ],
        scratch_shapes=[pltpu.VMEM((1, N_EXPERTS), jnp.float32)],
    )(xf, gate_w)

    return (gates.reshape(b, s, TOP_K), idx.reshape(b, s, TOP_K), aux[0, 0])


# speedup vs baseline: 1.4860x; 1.2386x over previous
import functools

import jax
import jax.numpy as jnp
from jax.experimental import pallas as pl
from jax.experimental.pallas import tpu as pltpu

D_MODEL = 4096
N_EXPERTS = 64
TOP_K = 8
AUX_W = 0.01
CHUNK = 128


def _router_body(x_ref, w_ref, gates_ref, idx_ref, aux_ref, counts_ref,
                 *, blk_t, n_blk, n_tokens):
    i = pl.program_id(0)
    lt = jax.lax.dot_general(
        w_ref[...], x_ref[...],
        dimension_numbers=(((1,), (1,)), ((), ())),
        preferred_element_type=jnp.float32,
    )

    sub_iota = jax.lax.broadcasted_iota(
        jnp.int32, (N_EXPERTS, CHUNK), 0).astype(jnp.float32)

    @pl.when(i == 0)
    def _init():
        counts_ref[...] = jnp.zeros_like(counts_ref)

    for c in range(blk_t // CHUNK):
        work = jax.lax.slice(lt, (0, c * CHUNK), (N_EXPERTS, (c + 1) * CHUNK))
        vals = []
        idxs = []
        for _ in range(TOP_K):
            mj = jnp.max(work, axis=0, keepdims=True)
            ij = jnp.min(jnp.where(work == mj, sub_iota, float(N_EXPERTS)),
                         axis=0, keepdims=True)
            vals.append(mj)
            idxs.append(ij)
            work = jnp.where(sub_iota == ij, -jnp.inf, work)

        v = jnp.concatenate(vals, axis=0)
        ev = jnp.exp(v - vals[0])
        g = ev / jnp.sum(ev, axis=0, keepdims=True)
        ix = jnp.concatenate(idxs, axis=0)

        gates_ref[pl.ds(c * CHUNK, CHUNK), :] = g.T
        idx_ref[pl.ds(c * CHUNK, CHUNK), :] = ix.T.astype(jnp.int32)

        counts_ref[...] += jnp.where(sub_iota == idxs[0], 1.0, 0.0)

    @pl.when(i == n_blk - 1)
    def _fin():
        freq = jnp.sum(counts_ref[...], axis=1, keepdims=True) / n_tokens
        diff = freq - (1.0 / N_EXPERTS)
        aux_ref[...] = AUX_W * N_EXPERTS * jnp.sum(diff * diff,
                                                   axis=(0, 1), keepdims=True)


def kernel(x, gate_w):
    b, s, d = x.shape
    n_tokens = b * s
    blk_t = 512
    n_blk = n_tokens // blk_t
    xf = x.reshape(n_tokens, d)

    gates, idx, aux = pl.pallas_call(
        functools.partial(_router_body, blk_t=blk_t, n_blk=n_blk,
                          n_tokens=n_tokens),
        grid=(n_blk,),
        in_specs=[
            pl.BlockSpec((blk_t, d), lambda i: (i, 0)),
            pl.BlockSpec((N_EXPERTS, d), lambda i: (0, 0)),
        ],
        out_specs=[
            pl.BlockSpec((blk_t, TOP_K), lambda i: (i, 0)),
            pl.BlockSpec((blk_t, TOP_K), lambda i: (i, 0)),
            pl.BlockSpec((1, 1), lambda i: (0, 0)),
        ],
        out_shape=[
            jax.ShapeDtypeStruct((n_tokens, TOP_K), jnp.float32),
            jax.ShapeDtypeStruct((n_tokens, TOP_K), jnp.int32),
            jax.ShapeDtypeStruct((1, 1), jnp.float32),
        ],
        scratch_shapes=[pltpu.VMEM((N_EXPERTS, CHUNK), jnp.float32)],
    )(xf, gate_w)

    return (gates.reshape(b, s, TOP_K), idx.reshape(b, s, TOP_K), aux[0, 0])


# speedup vs baseline: 1.5859x; 1.0672x over previous
import functools

import jax
import jax.numpy as jnp
from jax.experimental import pallas as pl
from jax.experimental.pallas import tpu as pltpu

D_MODEL = 4096
N_EXPERTS = 64
TOP_K = 8
AUX_W = 0.01
CHUNK = 128


def _router_body(x_ref, w_ref, gates_ref, idx_ref, aux_ref, counts_ref,
                 *, blk_t, n_blk, n_tokens):
    i = pl.program_id(0)
    lt = jax.lax.dot_general(
        w_ref[...], x_ref[...],
        dimension_numbers=(((1,), (1,)), ((), ())),
        preferred_element_type=jnp.float32,
    )

    sub_iota = jax.lax.broadcasted_iota(
        jnp.int32, (N_EXPERTS, CHUNK), 0).astype(jnp.float32)

    @pl.when(i == 0)
    def _init():
        counts_ref[...] = jnp.zeros_like(counts_ref)

    for c in range(blk_t // CHUNK):
        work = jax.lax.slice(lt, (0, c * CHUNK), (N_EXPERTS, (c + 1) * CHUNK))
        vals = []
        idxs = []
        for _ in range(TOP_K):
            mj = jnp.max(work, axis=0, keepdims=True)
            ij = jnp.min(jnp.where(work == mj, sub_iota, float(N_EXPERTS)),
                         axis=0, keepdims=True)
            vals.append(mj)
            idxs.append(ij)
            work = jnp.where(sub_iota == ij, -jnp.inf, work)

        v = jnp.concatenate(vals, axis=0)
        ev = jnp.exp(v - vals[0])
        g = ev / jnp.sum(ev, axis=0, keepdims=True)
        ix = jnp.concatenate(idxs, axis=0)

        gates_ref[pl.ds(c * CHUNK, CHUNK), :] = g.T
        idx_ref[pl.ds(c * CHUNK, CHUNK), :] = ix.T.astype(jnp.int32)

        counts_ref[...] += jnp.where(sub_iota == idxs[0], 1.0, 0.0)

    @pl.when(i == n_blk - 1)
    def _fin():
        freq = jnp.sum(counts_ref[...], axis=1, keepdims=True) / n_tokens
        diff = freq - (1.0 / N_EXPERTS)
        aux_ref[...] = AUX_W * N_EXPERTS * jnp.sum(diff * diff,
                                                   axis=(0, 1), keepdims=True)


def kernel(x, gate_w):
    b, s, d = x.shape
    n_tokens = b * s
    blk_t = 1024
    n_blk = n_tokens // blk_t
    xf = x.reshape(n_tokens, d)

    gates, idx, aux = pl.pallas_call(
        functools.partial(_router_body, blk_t=blk_t, n_blk=n_blk,
                          n_tokens=n_tokens),
        grid=(n_blk,),
        in_specs=[
            pl.BlockSpec((blk_t, d), lambda i: (i, 0)),
            pl.BlockSpec((N_EXPERTS, d), lambda i: (0, 0)),
        ],
        out_specs=[
            pl.BlockSpec((blk_t, TOP_K), lambda i: (i, 0)),
            pl.BlockSpec((blk_t, TOP_K), lambda i: (i, 0)),
            pl.BlockSpec((1, 1), lambda i: (0, 0)),
        ],
        out_shape=[
            jax.ShapeDtypeStruct((n_tokens, TOP_K), jnp.float32),
            jax.ShapeDtypeStruct((n_tokens, TOP_K), jnp.int32),
            jax.ShapeDtypeStruct((1, 1), jnp.float32),
        ],
        scratch_shapes=[pltpu.VMEM((N_EXPERTS, CHUNK), jnp.float32)],
    )(xf, gate_w)

    return (gates.reshape(b, s, TOP_K), idx.reshape(b, s, TOP_K), aux[0, 0])
